# baseline (speedup 1.0000x reference)
import functools

import jax
import jax.numpy as jnp
from jax import lax
from jax.experimental import pallas as pl
from jax.experimental.pallas import tpu as pltpu
from jax.experimental.pallas import tpu_sc as plsc

_B = 8192
_D = 2048
_E = 16
_M_BLK = 512

_NC = 2
_NS = 16
_NW = _NC * _NS
_ROWS_PER_W = _B // _NW
_GROUPS = _ROWS_PER_W // 16


def _matmul_body(x_ref, w_ref, out_ref):
    out_ref[...] = lax.dot_general(
        x_ref[...], w_ref[...],
        dimension_numbers=(((1,), (1,)), ((), ())),
        preferred_element_type=jnp.float32)


@jax.jit
def _logits_call(x, w):
    return pl.pallas_call(
        _matmul_body,
        grid=(_B // _M_BLK,),
        in_specs=[
            pl.BlockSpec((_M_BLK, _D), lambda i: (i, 0)),
            pl.BlockSpec((_E, _D), lambda i: (0, 0)),
        ],
        out_specs=pl.BlockSpec((_M_BLK, _E), lambda i: (i, 0)),
        out_shape=jax.ShapeDtypeStruct((_B, _E), jnp.float32),
        compiler_params=pltpu.CompilerParams(
            dimension_semantics=("arbitrary",)),
    )(x, w)


def _gate_body(logits_hbm, w_hbm, idx_hbm, logits_v, w_v, idx_v):
    wid = lax.axis_index("s") * _NC + lax.axis_index("c")
    base = wid * (_ROWS_PER_W * _E)
    obase = wid * (_ROWS_PER_W * 2)
    pltpu.sync_copy(logits_hbm.at[pl.ds(base, _ROWS_PER_W * _E)], logits_v)

    lanes = lax.iota(jnp.int32, 16)

    def group(g, carry):
        row_off = (g * 16 + lanes) * _E
        m1 = jnp.full((16,), -jnp.inf, jnp.float32)
        m2 = jnp.full((16,), -jnp.inf, jnp.float32)
        i1 = jnp.zeros((16,), jnp.int32)
        i2 = jnp.zeros((16,), jnp.int32)
        for e in range(_E):
            v = plsc.load_gather(logits_v, [row_off + e])
            ev = jnp.full((16,), e, jnp.int32)
            gt1 = v > m1
            gt2 = v > m2
            m2 = jnp.where(gt1, m1, jnp.where(gt2, v, m2))
            i2 = jnp.where(gt1, i1, jnp.where(gt2, ev, i2))
            m1 = jnp.where(gt1, v, m1)
            i1 = jnp.where(gt1, ev, i1)
        w1 = 1.0 / (1.0 + jnp.exp(m2 - m1))
        w2 = 1.0 - w1
        pos = (g * 16 + lanes) * 2
        plsc.store_scatter(w_v, [pos], w1)
        plsc.store_scatter(w_v, [pos + 1], w2)
        plsc.store_scatter(idx_v, [pos], i1)
        plsc.store_scatter(idx_v, [pos + 1], i2)
        return carry

    lax.fori_loop(0, _GROUPS, group, 0)

    pltpu.sync_copy(w_v, w_hbm.at[pl.ds(obase, _ROWS_PER_W * 2)])
    pltpu.sync_copy(idx_v, idx_hbm.at[pl.ds(obase, _ROWS_PER_W * 2)])


@jax.jit
def _gate_call(logits):
    f = pl.kernel(
        _gate_body,
        mesh=plsc.VectorSubcoreMesh(
            core_axis_name="c", subcore_axis_name="s"),
        out_type=[
            jax.ShapeDtypeStruct((_B * 2,), jnp.float32),
            jax.ShapeDtypeStruct((_B * 2,), jnp.int32),
        ],
        scratch_types=[
            pltpu.VMEM((_ROWS_PER_W * _E,), jnp.float32),
            pltpu.VMEM((_ROWS_PER_W * 2,), jnp.float32),
            pltpu.VMEM((_ROWS_PER_W * 2,), jnp.int32),
        ],
        compiler_params=pltpu.CompilerParams(needs_layout_passes=False),
    )
    w_flat, idx_flat = f(logits.reshape(-1))
    return w_flat.reshape(_B, 2), idx_flat.reshape(_B, 2)


def kernel(x, W_gate, W_noise):
    clean_logits = _logits_call(x, W_gate)
    combined_weights, top_k_indices = _gate_call(clean_logits)
    return (combined_weights, top_k_indices, clean_logits)


# speedup vs baseline: 1.1752x; 1.1752x over previous
import jax
import jax.numpy as jnp
from jax import lax
from jax.experimental import pallas as pl
from jax.experimental.pallas import tpu as pltpu
from jax.experimental.pallas import tpu_sc as plsc

_B = 8192
_D = 2048
_E = 16
_M_BLK = 1024

_NC = 2
_NS = 16
_NW = _NC * _NS
_ROWS_PER_W = _B // _NW
_GROUPS = _ROWS_PER_W // 16
_SC_ROWS_PER_W = _ROWS_PER_W // 8


def _matmul_body(x_ref, w_ref, out_ref):
    out_ref[...] = lax.dot_general(
        x_ref[...], w_ref[...],
        dimension_numbers=(((1,), (1,)), ((), ())),
        preferred_element_type=jnp.float32)


@jax.jit
def _logits_call(x, w):
    return pl.pallas_call(
        _matmul_body,
        grid=(_B // _M_BLK,),
        in_specs=[
            pl.BlockSpec((_M_BLK, _D), lambda i: (i, 0)),
            pl.BlockSpec((_E, _D), lambda i: (0, 0)),
        ],
        out_specs=pl.BlockSpec((_M_BLK, _E), lambda i: (i, 0)),
        out_shape=jax.ShapeDtypeStruct((_B, _E), jnp.float32),
        compiler_params=pltpu.CompilerParams(
            dimension_semantics=("arbitrary",)),
    )(x, w)


def _gate_body(logits_hbm, w_hbm, idx_hbm, logits_v, w_v, idx_v):
    wid = lax.axis_index("s") * _NC + lax.axis_index("c")
    base = wid * _ROWS_PER_W
    pltpu.sync_copy(logits_hbm.at[pl.ds(base, _ROWS_PER_W)], logits_v)

    lanes = lax.iota(jnp.int32, 16)
    zeros = jnp.zeros((16,), jnp.int32)
    ones = jnp.full((16,), 1, jnp.int32)

    def group(g, carry):
        rows = g * 16 + lanes
        m1 = jnp.full((16,), -jnp.inf, jnp.float32)
        m2 = jnp.full((16,), -jnp.inf, jnp.float32)
        i1 = jnp.zeros((16,), jnp.int32)
        i2 = jnp.zeros((16,), jnp.int32)
        for e in range(_E):
            v = plsc.load_gather(
                logits_v, [rows, jnp.full((16,), e, jnp.int32)])
            ev = jnp.full((16,), e, jnp.int32)
            gt1 = v > m1
            gt2 = v > m2
            m2 = jnp.where(gt1, m1, jnp.where(gt2, v, m2))
            i2 = jnp.where(gt1, i1, jnp.where(gt2, ev, i2))
            m1 = jnp.where(gt1, v, m1)
            i1 = jnp.where(gt1, ev, i1)
        w1 = 1.0 / (1.0 + jnp.exp(m2 - m1))
        w2 = 1.0 - w1
        plsc.store_scatter(w_v, [rows, zeros], w1)
        plsc.store_scatter(w_v, [rows, ones], w2)
        plsc.store_scatter(idx_v, [rows, zeros], i1)
        plsc.store_scatter(idx_v, [rows, ones], i2)
        return carry

    lax.fori_loop(0, _GROUPS, group, 0)

    pltpu.sync_copy(w_v, w_hbm.at[pl.ds(base, _ROWS_PER_W)])
    pltpu.sync_copy(idx_v, idx_hbm.at[pl.ds(base, _ROWS_PER_W)])


@jax.jit
def _gate_call(logits):
    f = pl.kernel(
        _gate_body,
        mesh=plsc.VectorSubcoreMesh(
            core_axis_name="c", subcore_axis_name="s"),
        out_type=[
            jax.ShapeDtypeStruct((_B, 2), jnp.float32),
            jax.ShapeDtypeStruct((_B, 2), jnp.int32),
        ],
        scratch_types=[
            pltpu.VMEM((_ROWS_PER_W, _E), jnp.float32),
            pltpu.VMEM((_ROWS_PER_W, 2), jnp.float32),
            pltpu.VMEM((_ROWS_PER_W, 2), jnp.int32),
        ],
        compiler_params=pltpu.CompilerParams(needs_layout_passes=False),
    )
    return f(logits)


def kernel(x, W_gate, W_noise):
    clean_logits = _logits_call(x, W_gate)
    combined_weights, top_k_indices = _gate_call(clean_logits)
    return (combined_weights, top_k_indices, clean_logits)
